# Initial kernel scaffold; baseline (speedup 1.0000x reference)
import jax
import jax.numpy as jnp
from jax.experimental import pallas as pl


def kernel(x, embed_table, lin_w, lin_b):
    raise NotImplementedError("write your pallas kernel here")


# speedup vs baseline: 9.4317x; 9.4317x over previous
import functools

import jax
import jax.numpy as jnp
from jax import lax
from jax.experimental import pallas as pl
from jax.experimental.pallas import tpu as pltpu
from jax.experimental.pallas import tpu_sc as plsc

D = 16
B = 16384
H = 200
H0, H1 = 104, 96
NBUF = 4

_info = plsc.get_sparse_core_info()
_NC, _NS = _info.num_cores, _info.num_subcores
NW = _NC * _NS
PW = B // NW


def _body(x_hbm, tab_hbm, w_hbm, b_hbm, out_hbm,
          idx_v, rows_v, tbuf, outv, w_v, b_v, s0, s1, s2, s3):
    c = lax.axis_index("c")
    s = lax.axis_index("s")
    wid = s * _NC + c
    base = wid * PW

    pltpu.sync_copy(x_hbm.at[pl.ds(base * H, PW * H)], idx_v)
    pltpu.sync_copy(w_hbm, w_v)
    pltpu.sync_copy(b_hbm, b_v)
    sems = (s0, s1, s2, s3)
    w_vec = w_v[...]
    b_vec = b_v[...]

    def issue(i, slot):
        off = i * H
        pltpu.async_copy(tab_hbm.at[idx_v.at[pl.ds(off, H0)]],
                         rows_v.at[slot, pl.ds(0, H0)], sems[slot])
        pltpu.async_copy(tab_hbm.at[idx_v.at[pl.ds(off + H0, H1)]],
                         rows_v.at[slot, pl.ds(H0, H1)], sems[slot])

    def wait(slot):
        pltpu.make_async_copy(tab_hbm.at[pl.ds(0, H)],
                              rows_v.at[slot], sems[slot]).wait()

    for e in range(NBUF):
        issue(e, e)

    def outer(k, carry):
        for e in range(NBUF):
            i = k * NBUF + e
            wait(e)

            def inner(j, accs):
                a0, a1, a2, a3 = accs
                jj = j * 4
                a0 = a0 + rows_v[e, jj]
                a1 = a1 + rows_v[e, jj + 1]
                a2 = a2 + rows_v[e, jj + 2]
                a3 = a3 + rows_v[e, jj + 3]
                return (a0, a1, a2, a3)

            zero = jnp.zeros((D,), jnp.float32)
            a0, a1, a2, a3 = lax.fori_loop(0, H // 4, inner,
                                           (zero, zero, zero, zero))
            acc = (a0 + a1) + (a2 + a3)
            tbuf[pl.ds(i * D, D)] = acc * w_vec

            @pl.when(i + NBUF < PW)
            def _():
                issue(i + NBUF, e)
        return carry

    lax.fori_loop(0, PW // NBUF, outer, 0)

    def finalize(g, carry):
        row_ids = (g * D + lax.iota(jnp.int32, D)) * D
        zv = jnp.zeros((D,), jnp.float32)
        for l in range(D):
            zv = zv + plsc.load_gather(tbuf, [row_ids + l])
        z = zv / jnp.float32(H) + b_vec
        y = 1.0 / (1.0 + jnp.exp(-z))
        v = y * 10000.0
        v = (v + 8388608.0) - 8388608.0
        outv[pl.ds(g * D, D)] = v / 10000.0
        return carry

    lax.fori_loop(0, PW // D, finalize, 0)
    pltpu.sync_copy(outv, out_hbm.at[pl.ds(base, PW)])


@functools.partial(jax.jit, static_argnames=())
def _run(x_flat, embed_table, w16, b16):
    mesh = plsc.VectorSubcoreMesh(core_axis_name="c", subcore_axis_name="s")
    f = functools.partial(
        pl.kernel,
        out_type=jax.ShapeDtypeStruct((B,), jnp.float32),
        mesh=mesh,
        compiler_params=pltpu.CompilerParams(needs_layout_passes=False,
                                             use_tc_tiling_on_sc=False),
        scratch_types=[
            pltpu.VMEM((PW * H,), jnp.int32),
            pltpu.VMEM((NBUF, H, D), jnp.float32),
            pltpu.VMEM((PW * D,), jnp.float32),
            pltpu.VMEM((PW,), jnp.float32),
            pltpu.VMEM((D,), jnp.float32),
            pltpu.VMEM((D,), jnp.float32),
            pltpu.SemaphoreType.DMA,
            pltpu.SemaphoreType.DMA,
            pltpu.SemaphoreType.DMA,
            pltpu.SemaphoreType.DMA,
        ],
    )(_body)
    return f(x_flat, embed_table, w16, b16)


def kernel(x, embed_table, lin_w, lin_b):
    x_flat = jnp.reshape(x, (-1,)).astype(jnp.int32)
    w16 = jnp.reshape(lin_w, (D,))
    b16 = jnp.broadcast_to(lin_b, (D,))
    y = _run(x_flat, embed_table, w16, b16)
    return jnp.reshape(y, (B, 1))


# speedup vs baseline: 23.2841x; 2.4687x over previous
import functools

import jax
import jax.numpy as jnp
from jax import lax
from jax.experimental import pallas as pl
from jax.experimental.pallas import tpu as pltpu
from jax.experimental.pallas import tpu_sc as plsc

V = 1000000
D = 16
B = 16384
H = 200
HP = 208
H0, H1 = 104, 96
NBUF = 8
BLK = 65536

_info = plsc.get_sparse_core_info()
_NC, _NS = _info.num_cores, _info.num_subcores
NW = _NC * _NS
PW = B // NW


def _proj_body(w_ref, t_ref, o_ref):
    o_ref[...] = jnp.sum(t_ref[...] * w_ref[...], axis=0, keepdims=True)


def _project(tab_t, w_col):
    grid = (V + BLK - 1) // BLK
    return pl.pallas_call(
        _proj_body,
        grid=(grid,),
        in_specs=[
            pl.BlockSpec((D, 1), lambda i: (0, 0)),
            pl.BlockSpec((D, BLK), lambda i: (0, i)),
        ],
        out_specs=pl.BlockSpec((1, BLK), lambda i: (0, i)),
        out_shape=jax.ShapeDtypeStruct((1, V), jnp.float32),
    )(w_col, tab_t)


def _sc_body(x_hbm, proj_hbm, b_hbm, out_hbm,
             idx_v, vals_v, tbuf, outv, b_v, *sems):
    c = lax.axis_index("c")
    s = lax.axis_index("s")
    wid = s * _NC + c
    base = wid * PW

    pltpu.sync_copy(x_hbm.at[pl.ds(base * H, PW * H)], idx_v)
    pltpu.sync_copy(b_hbm, b_v)
    b_vec = b_v[...]

    zero = jnp.zeros((D,), jnp.float32)
    for slot in range(NBUF):
        vals_v[slot, pl.ds(H - 8, D)] = jnp.zeros((D,), jnp.float32)

    def issue(i, slot):
        off = i * H
        pltpu.async_copy(proj_hbm.at[idx_v.at[pl.ds(off, H0)]],
                         vals_v.at[slot, pl.ds(0, H0)], sems[slot])
        pltpu.async_copy(proj_hbm.at[idx_v.at[pl.ds(off + H0, H1)]],
                         vals_v.at[slot, pl.ds(H0, H1)], sems[slot])

    def wait(slot):
        pltpu.make_async_copy(proj_hbm.at[pl.ds(0, H)],
                              vals_v.at[slot, pl.ds(0, H)],
                              sems[slot]).wait()

    for e in range(NBUF):
        issue(e, e)

    def outer(k, carry):
        for e in range(NBUF):
            i = k * NBUF + e
            wait(e)
            acc = zero
            for j in range(HP // D):
                acc = acc + vals_v[e, pl.ds(j * D, D)]
            tbuf[pl.ds(i * D, D)] = acc

            @pl.when(i + NBUF < PW)
            def _():
                issue(i + NBUF, e)
        return carry

    lax.fori_loop(0, PW // NBUF, outer, 0)

    def finalize(g, carry):
        row_ids = (g * D + lax.iota(jnp.int32, D)) * D
        zv = jnp.zeros((D,), jnp.float32)
        for l in range(D):
            zv = zv + plsc.load_gather(tbuf, [row_ids + l])
        z = zv / jnp.float32(H) + b_vec
        y = 1.0 / (1.0 + jnp.exp(-z))
        v = y * 10000.0
        v = (v + 8388608.0) - 8388608.0
        outv[pl.ds(g * D, D)] = v / 10000.0
        return carry

    lax.fori_loop(0, PW // D, finalize, 0)
    pltpu.sync_copy(outv, out_hbm.at[pl.ds(base, PW)])


@jax.jit
def _run(x_flat, proj, b16):
    mesh = plsc.VectorSubcoreMesh(core_axis_name="c", subcore_axis_name="s")
    f = functools.partial(
        pl.kernel,
        out_type=jax.ShapeDtypeStruct((B,), jnp.float32),
        mesh=mesh,
        compiler_params=pltpu.CompilerParams(needs_layout_passes=False,
                                             use_tc_tiling_on_sc=False),
        scratch_types=[
            pltpu.VMEM((PW * H,), jnp.int32),
            pltpu.VMEM((NBUF, HP), jnp.float32),
            pltpu.VMEM((PW * D,), jnp.float32),
            pltpu.VMEM((PW,), jnp.float32),
            pltpu.VMEM((D,), jnp.float32),
        ] + [pltpu.SemaphoreType.DMA] * NBUF,
    )(_sc_body)
    return f(x_flat, proj, b16)


def kernel(x, embed_table, lin_w, lin_b):
    x_flat = jnp.reshape(x, (-1,))
    tab_t = embed_table.T
    w_col = jnp.reshape(lin_w, (D, 1))
    b16 = jnp.broadcast_to(lin_b, (D,))
    proj = jnp.reshape(_project(tab_t, w_col), (V,))
    y = _run(x_flat, proj, b16)
    return jnp.reshape(y, (B, 1))


# speedup vs baseline: 28.1741x; 1.2100x over previous
import functools

import jax
import jax.numpy as jnp
from jax import lax
from jax.experimental import pallas as pl
from jax.experimental.pallas import tpu as pltpu
from jax.experimental.pallas import tpu_sc as plsc

V = 1000000
D = 16
B = 16384
H = 200
HP = 208
H0, H1 = 104, 96
NBUF = 8
BLK = 65536

_info = plsc.get_sparse_core_info()
_NC, _NS = _info.num_cores, _info.num_subcores
NW = _NC * _NS
PW = B // NW


def _proj_body(w_ref, t_ref, o_ref):
    o_ref[...] = jnp.sum(t_ref[...] * w_ref[...], axis=0)


def _project(tab_t, w_col):
    grid = (V + BLK - 1) // BLK
    return pl.pallas_call(
        _proj_body,
        grid=(grid,),
        in_specs=[
            pl.BlockSpec((D, 1), lambda i: (0, 0)),
            pl.BlockSpec((D, BLK), lambda i: (0, i)),
        ],
        out_specs=pl.BlockSpec((BLK,), lambda i: (i,)),
        out_shape=jax.ShapeDtypeStruct((V,), jnp.float32),
    )(w_col, tab_t)


def _sc_body(x_hbm, proj_hbm, b_hbm, out_hbm,
             idx_v, vals_v, tbuf, outv, b_v, *sems):
    c = lax.axis_index("c")
    s = lax.axis_index("s")
    wid = s * _NC + c
    base = wid * PW

    pltpu.sync_copy(x_hbm.at[pl.ds(base, PW)], idx_v)
    pltpu.sync_copy(b_hbm, b_v)
    b_vec = b_v[...]

    zero = jnp.zeros((D,), jnp.float32)
    for slot in range(NBUF):
        vals_v[slot, pl.ds(H - 8, D)] = jnp.zeros((D,), jnp.float32)

    def issue(i, slot):
        pltpu.async_copy(proj_hbm.at[idx_v.at[i, pl.ds(0, H0)]],
                         vals_v.at[slot, pl.ds(0, H0)], sems[slot])
        pltpu.async_copy(proj_hbm.at[idx_v.at[i, pl.ds(H0, H1)]],
                         vals_v.at[slot, pl.ds(H0, H1)], sems[slot])

    def wait(slot):
        pltpu.make_async_copy(proj_hbm.at[pl.ds(0, H)],
                              vals_v.at[slot, pl.ds(0, H)],
                              sems[slot]).wait()

    for e in range(NBUF):
        issue(e, e)

    def outer(k, carry):
        for e in range(NBUF):
            i = k * NBUF + e
            wait(e)
            acc = zero
            for j in range(HP // D):
                acc = acc + vals_v[e, pl.ds(j * D, D)]
            tbuf[pl.ds(i * D, D)] = acc

            @pl.when(i + NBUF < PW)
            def _():
                issue(i + NBUF, e)
        return carry

    lax.fori_loop(0, PW // NBUF, outer, 0)

    def finalize(g, carry):
        row_ids = (g * D + lax.iota(jnp.int32, D)) * D
        zv = jnp.zeros((D,), jnp.float32)
        for l in range(D):
            zv = zv + plsc.load_gather(tbuf, [row_ids + l])
        z = zv / jnp.float32(H) + b_vec
        y = 1.0 / (1.0 + jnp.exp(-z))
        v = y * 10000.0
        v = (v + 8388608.0) - 8388608.0
        outv[pl.ds(g * D, D)] = v / 10000.0
        return carry

    lax.fori_loop(0, PW // D, finalize, 0)
    pltpu.sync_copy(outv, out_hbm.at[pl.ds(base, PW)])


@jax.jit
def _run(x2d, proj, b16):
    mesh = plsc.VectorSubcoreMesh(core_axis_name="c", subcore_axis_name="s")
    f = functools.partial(
        pl.kernel,
        out_type=jax.ShapeDtypeStruct((B,), jnp.float32),
        mesh=mesh,
        compiler_params=pltpu.CompilerParams(needs_layout_passes=False,
                                             use_tc_tiling_on_sc=False),
        scratch_types=[
            pltpu.VMEM((PW, H), jnp.int32),
            pltpu.VMEM((NBUF, HP), jnp.float32),
            pltpu.VMEM((PW * D,), jnp.float32),
            pltpu.VMEM((PW,), jnp.float32),
            pltpu.VMEM((D,), jnp.float32),
        ] + [pltpu.SemaphoreType.DMA] * NBUF,
    )(_sc_body)
    return f(x2d, proj, b16)


def kernel(x, embed_table, lin_w, lin_b):
    tab_t = embed_table.T
    w_col = jnp.reshape(lin_w, (D, 1))
    b16 = jnp.broadcast_to(lin_b, (D,))
    proj = _project(tab_t, w_col)
    y = _run(x, proj, b16)
    return jnp.reshape(y, (B, 1))
